# Initial kernel scaffold; baseline (speedup 1.0000x reference)
import jax
import jax.numpy as jnp
from jax.experimental import pallas as pl


def kernel(input_ids, embedding, W, b):
    raise NotImplementedError("write your pallas kernel here")


# speedup vs baseline: 15.8939x; 15.8939x over previous
import functools

import jax
import jax.numpy as jnp
from jax import lax
from jax.experimental import pallas as pl
from jax.experimental.pallas import tpu as pltpu
from jax.experimental.pallas import tpu_sc as plsc

_BATCH = 4096
_SEQ = 200
_D = 64
_HALF = 100
_NCLS = 2
_NCHUNK = _D // 16


@functools.cache
def _build_pool():
    info = plsc.get_sparse_core_info()
    nc, ns = info.num_cores, info.num_subcores
    nw = nc * ns
    bpw = _BATCH // nw
    mesh = plsc.VectorSubcoreMesh(core_axis_name="c", subcore_axis_name="s")

    @functools.partial(
        pl.kernel,
        mesh=mesh,
        compiler_params=pltpu.CompilerParams(use_tc_tiling_on_sc=False),
        out_type=jax.ShapeDtypeStruct((_BATCH, _D), jnp.float32),
        scratch_types=[
            pltpu.VMEM((2 * bpw, _HALF), jnp.int32),
            pltpu.VMEM((_SEQ, _D), jnp.float32),
            pltpu.VMEM((_SEQ, _D), jnp.float32),
            pltpu.VMEM((bpw, _D), jnp.float32),
            pltpu.SemaphoreType.DMA,
            pltpu.SemaphoreType.DMA,
        ],
    )
    def pool(ids_hbm, table_hbm, x_hbm, idx_v, rows_a, rows_b, out_v, sem_a, sem_b):
        wid = lax.axis_index("s") * nc + lax.axis_index("c")
        pltpu.sync_copy(ids_hbm.at[wid], idx_v)
        scale = jnp.float32(1.0 / _SEQ)

        def start(buf, sem, j0):
            pltpu.async_copy(table_hbm.at[idx_v.at[j0]], buf.at[pl.ds(0, _HALF)], sem)
            pltpu.async_copy(
                table_hbm.at[idx_v.at[j0 + 1]], buf.at[pl.ds(_HALF, _HALF)], sem
            )

        def wait(buf, sem, j0):
            pltpu.make_async_copy(
                table_hbm.at[idx_v.at[j0]], buf.at[pl.ds(0, _HALF)], sem
            ).wait()
            pltpu.make_async_copy(
                table_hbm.at[idx_v.at[j0 + 1]], buf.at[pl.ds(_HALF, _HALF)], sem
            ).wait()

        def accum(buf, row):
            def tbody(t, accs):
                return tuple(
                    accs[c]
                    + buf[t, pl.ds(c * 16, 16)]
                    + buf[t + _HALF, pl.ds(c * 16, 16)]
                    for c in range(_NCHUNK)
                )

            accs = lax.fori_loop(
                0,
                _HALF,
                tbody,
                tuple(jnp.zeros((16,), jnp.float32) for _ in range(_NCHUNK)),
                unroll=2,
            )
            for c in range(_NCHUNK):
                out_v[row, pl.ds(c * 16, 16)] = accs[c] * scale

        start(rows_a, sem_a, 0)

        def body(i, carry):
            j_a = 4 * i
            start(rows_b, sem_b, j_a + 2)
            wait(rows_a, sem_a, j_a)
            accum(rows_a, 2 * i)
            j_next = jnp.minimum(j_a + 4, 2 * bpw - 2)
            start(rows_a, sem_a, j_next)
            wait(rows_b, sem_b, j_a + 2)
            accum(rows_b, 2 * i + 1)
            return carry

        lax.fori_loop(0, bpw // 2, body, 0)
        wait(rows_a, sem_a, 2 * bpw - 2)
        pltpu.sync_copy(out_v, x_hbm.at[pl.ds(wid * bpw, bpw)])

    return pool, nw, bpw


def _linear_body(x_ref, w_ref, b_ref, o_ref):
    o_ref[...] = (
        jnp.dot(x_ref[...], w_ref[...], preferred_element_type=jnp.float32)
        + b_ref[...]
    )


def _linear(x, w, b):
    return pl.pallas_call(
        _linear_body,
        out_shape=jax.ShapeDtypeStruct((_BATCH, _NCLS), jnp.float32),
    )(x, w, b.reshape(1, _NCLS))


def kernel(input_ids, embedding, W, b):
    pool, nw, bpw = _build_pool()
    ids = input_ids.astype(jnp.int32).reshape(nw, 2 * bpw, _HALF)
    x = pool(ids, embedding)
    logits = _linear(x, W, b)
    return (logits, x)
